# baseline (speedup 1.0000x reference)
import functools

import jax
import jax.numpy as jnp
from jax.experimental import pallas as pl
from jax.experimental.pallas import tpu as pltpu

C_IN = 96
H = 384
W = 384
N = H * W
TH_A = 8
TH_C = 16
TH_S = 24
_BIG = 1e30


def _dwconv_flat(y_ext, dw_ref, nchan, th):
    l_out = th * W
    l_ext = (th + 2) * W
    y_ext = y_ext.astype(jnp.bfloat16).astype(jnp.float32)
    dw = dw_ref[...]
    zcol = jnp.zeros((nchan, 1), jnp.float32)
    wpos = jax.lax.broadcasted_iota(jnp.int32, (1, l_ext), 1) % W
    yl = jnp.concatenate([zcol, y_ext[:, :-1]], axis=1) * (
        wpos != 0).astype(jnp.float32)
    yr = jnp.concatenate([y_ext[:, 1:], zcol], axis=1) * (
        wpos != W - 1).astype(jnp.float32)
    acc = jnp.zeros((nchan, l_out), jnp.float32)
    for dy in range(3):
        o = dy * W
        acc = acc + dw[:, 3 * dy : 3 * dy + 1] * yl[:, o : o + l_out]
        acc = acc + dw[:, 3 * dy + 1 : 3 * dy + 2] * y_ext[:, o : o + l_out]
        acc = acc + dw[:, 3 * dy + 2 : 3 * dy + 3] * yr[:, o : o + l_out]
    return acc


def _make_ext(x_mid, x_top, x_bot, ntiles):
    i = pl.program_id(0)
    tfac = jnp.where(i == 0, 0.0, 1.0).astype(jnp.float32)
    bfac = jnp.where(i == ntiles - 1, 0.0, 1.0).astype(jnp.float32)
    return jnp.concatenate(
        [x_top[...] * tfac, x_mid[...], x_bot[...] * bfac], axis=1)


def _qk_tile(x_mid, x_top, x_bot, wqk_ref, dw_ref, th):
    ext = _make_ext(x_mid, x_top, x_bot, H // th)
    y = jax.lax.dot_general(
        wqk_ref[...], ext, (((1,), (0,)), ((), ())),
        preferred_element_type=jnp.float32)
    return _dwconv_flat(y, dw_ref, 2 * C_IN, th)


def _pass_a_body(x_mid, x_top, x_bot, wqk_ref, dw_ref, nrm_ref, qk_ref):
    i = pl.program_id(0)
    qk = _qk_tile(x_mid, x_top, x_bot, wqk_ref, dw_ref, TH_A)
    qk_ref[...] = qk
    q = qk[:C_IN, :]
    k = qk[C_IN:, :]
    rsq = jnp.sum(q * q, axis=1).reshape(1, C_IN)
    rsk = jnp.sum(k * k, axis=1).reshape(1, C_IN)
    nrm_part = jnp.concatenate([rsq, rsk], axis=0)

    @pl.when(i == 0)
    def _():
        nrm_ref[...] = nrm_part

    @pl.when(i != 0)
    def _():
        nrm_ref[...] += nrm_part


def _pass_a2_body(qk_ref, nrm_ref, s_ref):
    i = pl.program_id(0)
    qk = qk_ref[...]
    den = jnp.maximum(jnp.sqrt(nrm_ref[...]), 1e-12)
    qh = qk[:C_IN, :] / den[0:1, :].reshape(C_IN, 1)
    kh = qk[C_IN:, :] / den[1:2, :].reshape(C_IN, 1)
    s_part = jax.lax.dot_general(
        qh, kh, (((1,), (1,)), ((), ())),
        preferred_element_type=jnp.float32)

    @pl.when(i == 0)
    def _():
        s_ref[...] = s_part

    @pl.when(i != 0)
    def _():
        s_ref[...] += s_part


def _pass_b_body(s_ref, temp_ref, aa_ref, a_ref):
    hper = C_IN // 2
    jlt = (jax.lax.broadcasted_iota(jnp.int32, (1, hper, hper), 2)
           < jax.lax.broadcasted_iota(jnp.int32, (1, hper, hper), 1))
    zblk = jnp.zeros((hper, hper), jnp.float32)
    a_ref[:hper, hper:] = zblk
    a_ref[hper:, :hper] = zblk
    for h in range(2):
        sl = slice(h * hper, (h + 1) * hper)
        attn = s_ref[sl, sl] * temp_ref[0, h]
        beats = ((attn[:, None, :] > attn[:, :, None])
                 | ((attn[:, None, :] == attn[:, :, None]) & jlt))
        rank = jnp.sum(beats.astype(jnp.float32), axis=2)
        acc = jnp.zeros((hper, hper), jnp.float32)
        for bi, kk in enumerate((hper // 2, hper * 2 // 3, hper * 3 // 4,
                                 hper * 4 // 5)):
            m = rank < kk
            amk = jnp.where(m, attn, -_BIG)
            rmax = jnp.max(amk, axis=1, keepdims=True)
            e = jnp.exp(amk - rmax) * m.astype(jnp.float32)
            sm = e / jnp.sum(e, axis=1, keepdims=True)
            acc = acc + aa_ref[0, bi] * sm
        a_ref[sl, sl] = acc


def _pass_c_body(x_mid, x_top, x_bot, wv_ref, dw_ref, a_ref, p_ref, out_ref):
    ext = _make_ext(x_mid, x_top, x_bot, H // TH_C)
    y = jax.lax.dot_general(
        wv_ref[...], ext, (((1,), (0,)), ((), ())),
        preferred_element_type=jnp.float32)
    v = _dwconv_flat(y, dw_ref, C_IN, TH_C)
    u = jax.lax.dot_general(
        a_ref[...], v, (((1,), (0,)), ((), ())),
        preferred_element_type=jnp.float32)
    out_ref[...] = jax.lax.dot_general(
        p_ref[...], u, (((1,), (0,)), ((), ())),
        preferred_element_type=jnp.float32)


@functools.partial(jax.jit, static_argnames=())
def kernel(x, w_qkv, w_dw, w_proj, temperature, a1, a2, a3, a4):
    xf = x.reshape(C_IN, N)
    wqk = w_qkv[: 2 * C_IN, :, 0, 0]
    wv = w_qkv[2 * C_IN :, :, 0, 0]
    dwqk = w_dw[: 2 * C_IN, 0].reshape(2 * C_IN, 9)
    dwv = w_dw[2 * C_IN :, 0].reshape(C_IN, 9)
    p = w_proj[:, :, 0, 0]
    temp = temperature.reshape(1, 2)
    aa = jnp.concatenate([a1, a2, a3, a4]).reshape(1, 4)

    def xspecs(th):
        return [pl.BlockSpec((C_IN, th * W), lambda i: (0, i)),
                pl.BlockSpec((C_IN, W),
                             lambda i: (0, jnp.maximum(i * th - 1, 0))),
                pl.BlockSpec((C_IN, W),
                             lambda i: (0, jnp.minimum((i + 1) * th, H - 1)))]
    full = lambda shape: pl.BlockSpec(shape, lambda i: (0, 0))

    nrm, qk_hbm = pl.pallas_call(
        _pass_a_body,
        grid=(H // TH_A,),
        in_specs=xspecs(TH_A) + [
                  full((2 * C_IN, C_IN)), full((2 * C_IN, 9))],
        out_specs=[full((2, C_IN)),
                   pl.BlockSpec((2 * C_IN, TH_A * W), lambda i: (0, i))],
        out_shape=[jax.ShapeDtypeStruct((2, C_IN), jnp.float32),
                   jax.ShapeDtypeStruct((2 * C_IN, N), jnp.float32)],
        compiler_params=pltpu.CompilerParams(
            dimension_semantics=("arbitrary",)),
    )(xf, xf, xf, wqk, dwqk)

    s = pl.pallas_call(
        _pass_a2_body,
        grid=(H // TH_S,),
        in_specs=[pl.BlockSpec((2 * C_IN, TH_S * W), lambda i: (0, i)),
                  full((2, C_IN))],
        out_specs=full((C_IN, C_IN)),
        out_shape=jax.ShapeDtypeStruct((C_IN, C_IN), jnp.float32),
        compiler_params=pltpu.CompilerParams(
            dimension_semantics=("arbitrary",)),
    )(qk_hbm, nrm)

    a_comb = pl.pallas_call(
        _pass_b_body,
        out_shape=jax.ShapeDtypeStruct((C_IN, C_IN), jnp.float32),
    )(s, temp, aa)

    out = pl.pallas_call(
        _pass_c_body,
        grid=(H // TH_C,),
        in_specs=xspecs(TH_C) + [
                  full((C_IN, C_IN)), full((C_IN, 9)),
                  full((C_IN, C_IN)), full((C_IN, C_IN))],
        out_specs=pl.BlockSpec((C_IN, TH_C * W), lambda i: (0, i)),
        out_shape=jax.ShapeDtypeStruct((C_IN, N), jnp.float32),
        compiler_params=pltpu.CompilerParams(
            dimension_semantics=("arbitrary",)),
    )(xf, xf, xf, wv, dwv, a_comb, p)

    return out.reshape(1, C_IN, H, W)
